# baseline (speedup 1.0000x reference)
import jax
import jax.numpy as jnp
import numpy as np
from jax.experimental import pallas as pl
from jax.experimental.pallas import tpu as pltpu

B, S, D = 1, 2048, 2048
H, HD, RD = 16, 192, 64
ND = HD - RD
QLR, OLR, G = 1024, 128, 4
WINDOW = 512
EPS = 1e-6
SCALE = HD ** -0.5
NEG = -1e30

BS = 128
KW = WINDOW + BS
NBLK = S // BS
NBUF = KW // BS + 1
HPG = H // G

_PERM = np.concatenate([np.arange(0, RD, 2), np.arange(1, RD, 2)])
_IDX_HD = np.concatenate([np.arange(ND), ND + _PERM])
_IDX_QROWS = (np.arange(H * HD).reshape(H, HD) // HD * HD +
              _IDX_HD[None, :]).reshape(-1)
_IDX_OCOLS = (np.arange(HPG * HD).reshape(HPG, HD) // HD * HD +
              _IDX_HD[None, :]).reshape(-1)

_R = np.arange(BS)[:, None]
_C = np.arange(KW)[None, :]
_BAND = np.where((_C > _R) & (_C <= _R + WINDOW), 0.0, NEG).astype(np.float32)


def _fused_kernel(x_ref, wqa_ref, qnw_ref, wqb_ref, wkv_ref, kvnw_ref,
                  cos_ref, sin_ref, band_ref, sinkcol_ref, woa_ref, wob_ref,
                  out_ref, kbuf, vbuf, qbuf):
    i = pl.program_id(0)

    @pl.when(i == 0)
    def _init():
        kbuf[...] = jnp.zeros_like(kbuf)
        vbuf[...] = jnp.zeros_like(vbuf)
        qbuf[...] = jnp.zeros_like(qbuf)

    kwin = kbuf[BS:, :]
    vwin = vbuf[BS:, :]
    qall = qbuf[...]
    l = jax.lax.dot_general(qall, kwin, (((1,), (1,)), ((), ())))
    l = (l.reshape(H, BS, KW) + band_ref[...][None]).reshape(H * BS, KW)
    colpos = (i - 1 - 4) * BS + jax.lax.broadcasted_iota(jnp.int32, (1, KW), 1)
    l = l + jnp.where(colpos >= 0, 0.0, NEG)
    s = sinkcol_ref[...]
    m = jnp.maximum(jnp.max(l, axis=-1, keepdims=True), s)
    p = jnp.exp(l - m)
    denom = jnp.sum(p, axis=-1, keepdims=True) + jnp.exp(s - m)
    o = jax.lax.dot_general(p, vwin, (((1,), (0,)), ((), ())))
    o = o / denom
    o_parts = []
    for g in range(G):
        acc = None
        for j in range(HPG):
            h = g * HPG + j
            w = woa_ref[g, :, j * HD:(j + 1) * HD]
            t = jax.lax.dot_general(o[h * BS:(h + 1) * BS, :], w,
                                    (((1,), (1,)), ((), ())))
            acc = t if acc is None else acc + t
        o_parts.append(acc)
    o_lat = jnp.concatenate(o_parts, axis=-1)
    out_ref[...] = jax.lax.dot_general(
        o_lat, wob_ref[...], (((1,), (1,)), ((), ())))

    x = x_ref[...]
    ql = jax.lax.dot_general(x, wqa_ref[...], (((1,), (1,)), ((), ())))
    var = jnp.mean(ql * ql, axis=-1, keepdims=True)
    ql = ql * jax.lax.rsqrt(var + EPS) * qnw_ref[...]
    q = jax.lax.dot_general(ql, wqb_ref[...], (((1,), (1,)), ((), ())))
    q = q * SCALE

    kv = jax.lax.dot_general(x, wkv_ref[...], (((1,), (1,)), ((), ())))
    var2 = jnp.mean(kv * kv, axis=-1, keepdims=True)
    kv = kv * jax.lax.rsqrt(var2 + EPS) * kvnw_ref[...]

    cos = cos_ref[...]
    sin = sin_ref[...]
    k1 = kv[:, ND:ND + RD // 2]
    k2 = kv[:, ND + RD // 2:]
    kf = jnp.concatenate(
        [kv[:, :ND], k1 * cos - k2 * sin, k1 * sin + k2 * cos], axis=-1)
    kbuf[:KW, :] = kwin
    kbuf[KW:, :] = kf
    vbuf[:KW, :] = vwin
    vbuf[KW:, :] = kv
    for h in range(H):
        qh = q[:, h * HD:(h + 1) * HD]
        q1 = qh[:, ND:ND + RD // 2]
        q2 = qh[:, ND + RD // 2:]
        qbuf[h * BS:(h + 1) * BS, :] = jnp.concatenate(
            [qh[:, :ND], q1 * cos - q2 * sin, q1 * sin + q2 * cos], axis=-1)


def kernel(hidden_states, positions, wq_a, q_norm_w, wq_b, wkv, kv_norm_w,
           wo_a, wo_b, attn_sink):
    x = hidden_states.reshape(S, D)

    wqb_perm = wq_b[_IDX_QROWS, :]
    wkv_perm = wkv[_IDX_HD, :]
    kvnw_perm = kv_norm_w[_IDX_HD]
    woa_perm = wo_a[:, _IDX_OCOLS].reshape(G, OLR, HPG * HD)
    sinkcol = jnp.repeat(attn_sink, BS)[:, None]

    inv_freq = 1.0 / (10000.0 ** (np.arange(0, RD, 2, dtype=np.float32) / RD))
    ang = positions.astype(jnp.float32)[:, None] * inv_freq[None, :]
    cos = jnp.cos(ang)
    sin = jnp.sin(ang)
    band = jnp.asarray(_BAND)

    full = lambda shape: pl.BlockSpec(shape, lambda i: tuple(0 for _ in shape))
    blk = lambda i: (jnp.minimum(i, NBLK - 1), 0)
    out = pl.pallas_call(
        _fused_kernel,
        grid=(NBLK + 1,),
        in_specs=[
            pl.BlockSpec((BS, D), blk),
            full((QLR, D)),
            full((1, QLR)),
            full((H * HD, QLR)),
            full((HD, D)),
            full((1, HD)),
            pl.BlockSpec((BS, RD // 2), blk),
            pl.BlockSpec((BS, RD // 2), blk),
            full((BS, KW)),
            full((H * BS, 1)),
            full((G, OLR, HPG * HD)),
            full((D, G * OLR)),
        ],
        out_specs=pl.BlockSpec((BS, D), lambda i: (jnp.maximum(i - 1, 0), 0)),
        out_shape=jax.ShapeDtypeStruct((S, D), jnp.float32),
        scratch_shapes=[
            pltpu.VMEM((NBUF * BS, HD), jnp.float32),
            pltpu.VMEM((NBUF * BS, HD), jnp.float32),
            pltpu.VMEM((H * BS, HD), jnp.float32),
        ],
    )(x, wq_a, q_norm_w.reshape(1, QLR), wqb_perm, wkv_perm,
      kvnw_perm.reshape(1, HD), cos, sin, band, sinkcol, woa_perm, wo_b)

    return out.reshape(B, S, D)


# speedup vs baseline: 1.2785x; 1.2785x over previous
import jax
import jax.numpy as jnp
import numpy as np
from jax.experimental import pallas as pl
from jax.experimental.pallas import tpu as pltpu

B, S, D = 1, 2048, 2048
H, HD, RD = 16, 192, 64
ND = HD - RD
QLR, OLR, G = 1024, 128, 4
WINDOW = 512
EPS = 1e-6
SCALE = HD ** -0.5
NEG = -1e30

BP = 256
BQ = 256
KW = WINDOW + BQ
HPG = H // G

_PERM = np.concatenate([np.arange(0, RD, 2), np.arange(1, RD, 2)])
_IDX_HD = np.concatenate([np.arange(ND), ND + _PERM])
_IDX_QROWS = (np.arange(H * HD).reshape(H, HD) // HD * HD +
              _IDX_HD[None, :]).reshape(-1)
_IDX_OCOLS = (np.arange(HPG * HD).reshape(HPG, HD) // HD * HD +
              _IDX_HD[None, :]).reshape(-1)

_R = np.arange(BQ)[:, None]
_C = np.arange(KW)[None, :]
_BIAS3 = np.stack([
    np.where(_C <= _R, 0.0, NEG),
    np.where((_C <= _R + BQ) & (_C > _R + BQ - WINDOW), 0.0, NEG),
    np.where((_C > _R) & (_C <= _R + WINDOW), 0.0, NEG),
]).astype(np.float32)


def _proj_kernel(x_ref, wqa_ref, qnw_ref, wqb_ref, wkv_ref, kvnw_ref,
                 cos_ref, sin_ref, qf_ref, kf_ref, v_ref):
    x = x_ref[...]
    ql = jax.lax.dot_general(x, wqa_ref[...], (((1,), (1,)), ((), ())))
    var = jnp.mean(ql * ql, axis=-1, keepdims=True)
    ql = ql * jax.lax.rsqrt(var + EPS) * qnw_ref[...]
    q = jax.lax.dot_general(ql, wqb_ref[...], (((1,), (1,)), ((), ())))
    q = q * SCALE

    kv = jax.lax.dot_general(x, wkv_ref[...], (((1,), (1,)), ((), ())))
    var2 = jnp.mean(kv * kv, axis=-1, keepdims=True)
    kv = kv * jax.lax.rsqrt(var2 + EPS) * kvnw_ref[...]

    cos = cos_ref[...]
    sin = sin_ref[...]
    k1 = kv[:, ND:ND + RD // 2]
    k2 = kv[:, ND + RD // 2:]
    kf_ref[:, :ND] = kv[:, :ND].astype(jnp.bfloat16)
    kf_ref[:, ND:ND + RD // 2] = (k1 * cos - k2 * sin).astype(jnp.bfloat16)
    kf_ref[:, ND + RD // 2:] = (k1 * sin + k2 * cos).astype(jnp.bfloat16)
    v_ref[...] = kv.astype(jnp.bfloat16)
    for h in range(H):
        qh = q[:, h * HD:(h + 1) * HD]
        q1 = qh[:, ND:ND + RD // 2]
        q2 = qh[:, ND + RD // 2:]
        qf_ref[h, :, :ND] = qh[:, :ND].astype(jnp.bfloat16)
        qf_ref[h, :, ND:ND + RD // 2] = (
            q1 * cos - q2 * sin).astype(jnp.bfloat16)
        qf_ref[h, :, ND + RD // 2:] = (
            q1 * sin + q2 * cos).astype(jnp.bfloat16)


def _attn_kernel(qf_ref, kf_ref, v_ref, bias_ref, sinkcol_ref, woa_ref,
                 wob_ref, out_ref):
    i = pl.program_id(0)
    qb = i * BQ
    kstart = pl.multiple_of(jnp.maximum(qb - WINDOW, 0), BQ)
    kwin = kf_ref[pl.ds(kstart, KW), :]
    vwin = v_ref[pl.ds(kstart, KW), :].astype(jnp.float32)
    s = sinkcol_ref[...]

    qall = qf_ref[...].reshape(H * BQ, HD)
    l = jax.lax.dot_general(qall, kwin, (((1,), (1,)), ((), ())),
                            preferred_element_type=jnp.float32)
    l = (l.reshape(H, BQ, KW) + bias_ref[...]).reshape(H * BQ, KW)
    m = jnp.max(l, axis=-1, keepdims=True)
    m2 = jnp.maximum(m, s)
    p = jnp.exp(l - m2)
    denom = jnp.sum(p, axis=-1, keepdims=True) + jnp.exp(s - m2)
    o = jax.lax.dot_general(p, vwin, (((1,), (0,)), ((), ())))
    o = (o / denom).reshape(H, BQ, HD)

    o_parts = []
    for g in range(G):
        acc = None
        for j in range(HPG):
            h = g * HPG + j
            w = woa_ref[g, :, j * HD:(j + 1) * HD]
            t = jax.lax.dot_general(o[h], w, (((1,), (1,)), ((), ())))
            acc = t if acc is None else acc + t
        o_parts.append(acc)
    o_lat = jnp.concatenate(o_parts, axis=-1)
    out_ref[...] = jax.lax.dot_general(
        o_lat, wob_ref[...], (((1,), (1,)), ((), ())))


def kernel(hidden_states, positions, wq_a, q_norm_w, wq_b, wkv, kv_norm_w,
           wo_a, wo_b, attn_sink):
    x = hidden_states.reshape(S, D)

    wqb_perm = wq_b[_IDX_QROWS, :]
    wkv_perm = wkv[_IDX_HD, :]
    kvnw_perm = kv_norm_w[_IDX_HD]
    woa_perm = wo_a[:, _IDX_OCOLS].reshape(G, OLR, HPG * HD)
    sinkcol = jnp.repeat(attn_sink, BQ)[:, None]

    inv_freq = 1.0 / (10000.0 ** (np.arange(0, RD, 2, dtype=np.float32) / RD))
    ang = positions.astype(jnp.float32)[:, None] * inv_freq[None, :]
    cos = jnp.cos(ang)
    sin = jnp.sin(ang)
    bias3 = jnp.asarray(_BIAS3)

    full = lambda shape: pl.BlockSpec(shape, lambda i: tuple(0 for _ in shape))
    qf, kf, v = pl.pallas_call(
        _proj_kernel,
        grid=(S // BP,),
        in_specs=[
            pl.BlockSpec((BP, D), lambda i: (i, 0)),
            full((QLR, D)),
            full((1, QLR)),
            full((H * HD, QLR)),
            full((HD, D)),
            full((1, HD)),
            pl.BlockSpec((BP, RD // 2), lambda i: (i, 0)),
            pl.BlockSpec((BP, RD // 2), lambda i: (i, 0)),
        ],
        out_specs=[
            pl.BlockSpec((H, BP, HD), lambda i: (0, i, 0)),
            pl.BlockSpec((BP, HD), lambda i: (i, 0)),
            pl.BlockSpec((BP, HD), lambda i: (i, 0)),
        ],
        out_shape=[
            jax.ShapeDtypeStruct((H, S, HD), jnp.bfloat16),
            jax.ShapeDtypeStruct((S, HD), jnp.bfloat16),
            jax.ShapeDtypeStruct((S, HD), jnp.bfloat16),
        ],
    )(x, wq_a, q_norm_w.reshape(1, QLR), wqb_perm, wkv_perm,
      kvnw_perm.reshape(1, HD), cos, sin)

    out = pl.pallas_call(
        _attn_kernel,
        grid=(S // BQ,),
        in_specs=[
            pl.BlockSpec((H, BQ, HD), lambda i: (0, i, 0)),
            full((S, HD)),
            full((S, HD)),
            pl.BlockSpec((1, BQ, KW), lambda i: (jnp.minimum(i, 2), 0, 0)),
            full((H * BQ, 1)),
            full((G, OLR, HPG * HD)),
            full((D, G * OLR)),
        ],
        out_specs=pl.BlockSpec((BQ, D), lambda i: (i, 0)),
        out_shape=jax.ShapeDtypeStruct((S, D), jnp.float32),
    )(qf, kf, v, bias3, sinkcol, woa_perm, wo_b)

    return out.reshape(B, S, D)


# speedup vs baseline: 1.2932x; 1.0115x over previous
import jax
import jax.numpy as jnp
import numpy as np
from jax.experimental import pallas as pl
from jax.experimental.pallas import tpu as pltpu

B, S, D = 1, 2048, 2048
H, HD, RD = 16, 192, 64
ND = HD - RD
QLR, OLR, G = 1024, 128, 4
WINDOW = 512
EPS = 1e-6
SCALE = HD ** -0.5
NEG = -1e30

BP = 512
BQ = 256
KW = WINDOW + BQ
HPG = H // G

_PERM = np.concatenate([np.arange(0, RD, 2), np.arange(1, RD, 2)])
_IDX_HD = np.concatenate([np.arange(ND), ND + _PERM])
_IDX_QROWS = (np.arange(H * HD).reshape(H, HD) // HD * HD +
              _IDX_HD[None, :]).reshape(-1)
_IDX_OCOLS = (np.arange(HPG * HD).reshape(HPG, HD) // HD * HD +
              _IDX_HD[None, :]).reshape(-1)

_R = np.arange(BQ)[:, None]
_C = np.arange(KW)[None, :]
_BIAS3 = np.stack([
    np.where(_C <= _R, 0.0, NEG),
    np.where((_C <= _R + BQ) & (_C > _R + BQ - WINDOW), 0.0, NEG),
    np.where((_C > _R) & (_C <= _R + WINDOW), 0.0, NEG),
]).astype(np.float32)


def _proj_kernel(x_ref, wqa_ref, qnw_ref, wqb_ref, wkv_ref, kvnw_ref,
                 cos_ref, sin_ref, qf_ref, kf_ref, v_ref):
    x = x_ref[...]
    ql = jax.lax.dot_general(x, wqa_ref[...], (((1,), (1,)), ((), ())))
    var = jnp.mean(ql * ql, axis=-1, keepdims=True)
    ql = ql * jax.lax.rsqrt(var + EPS) * qnw_ref[...]
    q = jax.lax.dot_general(ql, wqb_ref[...], (((1,), (1,)), ((), ())))
    q = q * SCALE

    kv = jax.lax.dot_general(x, wkv_ref[...], (((1,), (1,)), ((), ())))
    var2 = jnp.mean(kv * kv, axis=-1, keepdims=True)
    kv = kv * jax.lax.rsqrt(var2 + EPS) * kvnw_ref[...]

    cos = cos_ref[...]
    sin = sin_ref[...]
    k1 = kv[:, ND:ND + RD // 2]
    k2 = kv[:, ND + RD // 2:]
    kf_ref[:, :ND] = kv[:, :ND].astype(jnp.bfloat16)
    kf_ref[:, ND:ND + RD // 2] = (k1 * cos - k2 * sin).astype(jnp.bfloat16)
    kf_ref[:, ND + RD // 2:] = (k1 * sin + k2 * cos).astype(jnp.bfloat16)
    v_ref[...] = kv.astype(jnp.bfloat16)
    for h in range(H):
        qh = q[:, h * HD:(h + 1) * HD]
        q1 = qh[:, ND:ND + RD // 2]
        q2 = qh[:, ND + RD // 2:]
        qf_ref[h, :, :ND] = qh[:, :ND].astype(jnp.bfloat16)
        qf_ref[h, :, ND:ND + RD // 2] = (
            q1 * cos - q2 * sin).astype(jnp.bfloat16)
        qf_ref[h, :, ND + RD // 2:] = (
            q1 * sin + q2 * cos).astype(jnp.bfloat16)


def _attn_kernel(qf_ref, kf_ref, v_ref, bias_ref, sinkcol_ref, woa_ref,
                 wob_ref, out_ref):
    i = pl.program_id(0)
    qb = i * BQ
    kstart = pl.multiple_of(jnp.maximum(qb - WINDOW, 0), BQ)
    kwin = kf_ref[pl.ds(kstart, KW), :]
    vwin = v_ref[pl.ds(kstart, KW), :].astype(jnp.float32)
    s = sinkcol_ref[...]

    qall = qf_ref[...].reshape(H * BQ, HD)
    l = jax.lax.dot_general(qall, kwin, (((1,), (1,)), ((), ())),
                            preferred_element_type=jnp.float32)
    l = (l.reshape(H, BQ, KW) + bias_ref[...]).reshape(H * BQ, KW)
    m = jnp.max(l, axis=-1, keepdims=True)
    m2 = jnp.maximum(m, s)
    p = jnp.exp(l - m2)
    denom = jnp.sum(p, axis=-1, keepdims=True) + jnp.exp(s - m2)
    o = jax.lax.dot_general(p, vwin, (((1,), (0,)), ((), ())))
    o = (o / denom).reshape(H, BQ, HD)

    o_parts = []
    for g in range(G):
        acc = None
        for j in range(HPG):
            h = g * HPG + j
            w = woa_ref[g, :, j * HD:(j + 1) * HD]
            t = jax.lax.dot_general(o[h], w, (((1,), (1,)), ((), ())))
            acc = t if acc is None else acc + t
        o_parts.append(acc)
    o_lat = jnp.concatenate(o_parts, axis=-1)
    out_ref[...] = jax.lax.dot_general(
        o_lat, wob_ref[...], (((1,), (1,)), ((), ())))


def kernel(hidden_states, positions, wq_a, q_norm_w, wq_b, wkv, kv_norm_w,
           wo_a, wo_b, attn_sink):
    x = hidden_states.reshape(S, D)

    wqb_perm = wq_b[_IDX_QROWS, :]
    wkv_perm = wkv[_IDX_HD, :]
    kvnw_perm = kv_norm_w[_IDX_HD]
    woa_perm = wo_a[:, _IDX_OCOLS].reshape(G, OLR, HPG * HD)
    sinkcol = jnp.repeat(attn_sink, BQ)[:, None]

    inv_freq = 1.0 / (10000.0 ** (np.arange(0, RD, 2, dtype=np.float32) / RD))
    ang = positions.astype(jnp.float32)[:, None] * inv_freq[None, :]
    cos = jnp.cos(ang)
    sin = jnp.sin(ang)
    bias3 = jnp.asarray(_BIAS3)

    full = lambda shape: pl.BlockSpec(shape, lambda i: tuple(0 for _ in shape))
    qf, kf, v = pl.pallas_call(
        _proj_kernel,
        grid=(S // BP,),
        in_specs=[
            pl.BlockSpec((BP, D), lambda i: (i, 0)),
            full((QLR, D)),
            full((1, QLR)),
            full((H * HD, QLR)),
            full((HD, D)),
            full((1, HD)),
            pl.BlockSpec((BP, RD // 2), lambda i: (i, 0)),
            pl.BlockSpec((BP, RD // 2), lambda i: (i, 0)),
        ],
        out_specs=[
            pl.BlockSpec((H, BP, HD), lambda i: (0, i, 0)),
            pl.BlockSpec((BP, HD), lambda i: (i, 0)),
            pl.BlockSpec((BP, HD), lambda i: (i, 0)),
        ],
        out_shape=[
            jax.ShapeDtypeStruct((H, S, HD), jnp.bfloat16),
            jax.ShapeDtypeStruct((S, HD), jnp.bfloat16),
            jax.ShapeDtypeStruct((S, HD), jnp.bfloat16),
        ],
    )(x, wq_a, q_norm_w.reshape(1, QLR), wqb_perm, wkv_perm,
      kvnw_perm.reshape(1, HD), cos, sin)

    out = pl.pallas_call(
        _attn_kernel,
        grid=(S // BQ,),
        in_specs=[
            pl.BlockSpec((H, BQ, HD), lambda i: (0, i, 0)),
            full((S, HD)),
            full((S, HD)),
            pl.BlockSpec((1, BQ, KW), lambda i: (jnp.minimum(i, 2), 0, 0)),
            full((H * BQ, 1)),
            full((G, OLR, HPG * HD)),
            full((D, G * OLR)),
        ],
        out_specs=pl.BlockSpec((BQ, D), lambda i: (i, 0)),
        out_shape=jax.ShapeDtypeStruct((S, D), jnp.float32),
    )(qf, kf, v, bias3, sinkcol, woa_perm, wo_b)

    return out.reshape(B, S, D)
